# baseline (speedup 1.0000x reference)
import functools
import math

import jax
import jax.numpy as jnp
from jax import lax
from jax.experimental import pallas as pl
from jax.experimental.pallas import tpu as pltpu
from jax.experimental.pallas import tpu_sc as plsc

_D = 64
_B = 4096
_L = 200
_V = 1000000
_N = _B * _L
_NC = 2
_NS = 16
_NW = _NC * _NS
_SCALE = math.sqrt(_D)

_VT = _V // 128
_VMAIN = _VT * 128
_T_PER_W = _VT // _NW
_T_EXTRA = _VT - _T_PER_W * _NW

_LPB = 2
_NBLK = _L // _LPB


def _iota16():
    return lax.iota(jnp.int32, 16)


def _k1_body(wt, wtail, tbl, in_v, out_v, tail_v, sem0, sem1):
    wid = lax.axis_index("s") * _NC + lax.axis_index("c")
    start = wid * _T_PER_W + jnp.minimum(wid, _T_EXTRA)
    cnt = _T_PER_W + jnp.where(wid < _T_EXTRA, 1, 0)
    sems = (sem0, sem1)

    perm = []
    for j in range(8):
        v = j * 16 + _iota16()
        perm.append((v // 2, (v % 2) * 64))

    def issue(j, sem, b):
        pltpu.async_copy(wt.at[:, pl.ds(j * 128, 128)], in_v.at[b], sem)

    issue(start, sems[0], 0)

    @pl.loop(0, (_T_PER_W + 2) // 2 + 1)
    def _pair(p):
        for b in range(2):
            k = p * 2 + b

            @pl.when(k < cnt)
            def _(b=b, k=k):
                j = start + k
                pltpu.make_async_copy(
                    wt.at[:, pl.ds(j * 128, 128)], in_v.at[b], sems[b]
                ).wait()

                @pl.when(k + 1 < cnt)
                def _():
                    issue(j + 1, sems[1 - b], 1 - b)

                @plsc.parallel_loop(0, 64, 1, unroll=2)
                def _tp(c, _b=b):
                    for jj in range(8):
                        d0, d1 = perm[jj]
                        vals = in_v[_b, c, pl.ds(jj * 16, 16)] * _SCALE
                        plsc.store_scatter(out_v.at[_b], [d0, d1 + c], vals)

                pltpu.sync_copy(out_v.at[b], tbl.at[pl.ds(j * 64, 64)])

    @pl.when(wid == _NW - 1)
    def _tail():
        pltpu.sync_copy(wtail, tail_v)

        @plsc.parallel_loop(0, 64, 1, unroll=2)
        def _tp(c):
            for jj in range(4):
                d0, d1 = perm[jj]
                vals = tail_v[c, pl.ds(jj * 16, 16)] * _SCALE
                plsc.store_scatter(out_v.at[0], [d0, d1 + c], vals)

        pltpu.sync_copy(
            out_v.at[0, pl.ds(0, 32)], tbl.at[pl.ds(_VMAIN // 2, 32)]
        )


@functools.partial(
    pl.kernel,
    out_type=jax.ShapeDtypeStruct((_V // 2, 128), jnp.float32),
    mesh=plsc.VectorSubcoreMesh(core_axis_name="c", subcore_axis_name="s"),
    scratch_types=[
        pltpu.VMEM((2, _D, 128), jnp.float32),
        pltpu.VMEM((2, _D, 128), jnp.float32),
        pltpu.VMEM((_D, _D), jnp.float32),
        pltpu.SemaphoreType.DMA,
        pltpu.SemaphoreType.DMA,
    ],
    compiler_params=pltpu.CompilerParams(
        use_tc_tiling_on_sc=True, needs_layout_passes=False
    ),
)
def _repack(wt, wtail, tbl, in_v, out_v, tail_v, sem0, sem1):
    _k1_body(wt, wtail, tbl, in_v, out_v, tail_v, sem0, sem1)


def _k2_body(tbl, idxt, out, idx_v, rows_v, out_v, sem0, sem1, semw):
    wid = lax.axis_index("s") * _NC + lax.axis_index("c")
    sems = (sem0, sem1)

    perm = []
    for u in range(_LPB):
        pu = []
        for j in range(4):
            c = j * 16 + _iota16()
            pu.append(u * 8192 + (c // 8) * 1024 + (c % 8) * 128)
        perm.append(pu)

    def issue(blk, b):
        l0 = blk * _LPB
        pltpu.sync_copy(
            idxt.at[pl.ds(l0, _LPB), pl.ds(wid * 128, 128)], idx_v.at[b]
        )
        for u in range(_LPB):
            pltpu.async_copy(
                tbl.at[idx_v.at[b, u]],
                rows_v.at[b, pl.ds(u * 128, 128)],
                sems[b],
            )

    def drain_writes(blk, b):
        l0 = blk * _LPB
        for u in range(_LPB):
            for ci in range(8):
                pltpu.make_async_copy(
                    out_v.at[b, pl.ds(u * 8192 + ci * 1024, 1024)],
                    out.at[l0 + u, ci, wid],
                    semw,
                ).wait()

    issue(jnp.int32(0), 0)

    @pl.loop(0, _NBLK // 2)
    def _pair(p):
        for b in range(2):
            blk = p * 2 + b

            for u in range(_LPB):
                pltpu.make_async_copy(
                    tbl.at[idx_v.at[b, u]],
                    rows_v.at[b, pl.ds(u * 128, 128)],
                    sems[b],
                ).wait()

            @pl.when(blk + 1 < _NBLK)
            def _(b=b, blk=blk):
                issue(blk + 1, 1 - b)

            @pl.when(blk >= 2)
            def _(b=b, blk=blk):
                drain_writes(blk - 2, b)

            for u in range(_LPB):

                @plsc.parallel_loop(0, 128, 1, unroll=2)
                def _tp(rr, _b=b, _u=u):
                    for jj in range(4):
                        vals = rows_v[_b, _u * 128 + rr, pl.ds(jj * 16, 16)]
                        plsc.store_scatter(
                            out_v.at[_b], [perm[_u][jj] + rr], vals
                        )

            l0 = blk * _LPB
            for u in range(_LPB):
                for ci in range(8):
                    pltpu.async_copy(
                        out_v.at[b, pl.ds(u * 8192 + ci * 1024, 1024)],
                        out.at[l0 + u, ci, wid],
                        semw,
                    )

    @pl.loop(0, 1)
    def _fin(_):
        for blkf in (_NBLK - 2, _NBLK - 1):
            drain_writes(jnp.int32(blkf), blkf % 2)


@functools.partial(
    pl.kernel,
    out_type=jax.ShapeDtypeStruct((_L, 8, _NW, 1024), jnp.float32),
    mesh=plsc.VectorSubcoreMesh(core_axis_name="c", subcore_axis_name="s"),
    scratch_types=[
        pltpu.VMEM((2, _LPB, 128), jnp.int32),
        pltpu.VMEM((2, _LPB * 128, _D), jnp.float32),
        pltpu.VMEM((2, _LPB * 8192), jnp.float32),
        pltpu.SemaphoreType.DMA,
        pltpu.SemaphoreType.DMA,
        pltpu.SemaphoreType.DMA,
    ],
    compiler_params=pltpu.CompilerParams(
        use_tc_tiling_on_sc=False, needs_layout_passes=False
    ),
)
def _gather(tbl, idxt, out, idx_v, rows_v, out_v, sem0, sem1, semw):
    _k2_body(tbl, idxt, out, idx_v, rows_v, out_v, sem0, sem1, semw)


def kernel(x, embedding_weight):
    wt = embedding_weight.T
    wtail = embedding_weight[_VMAIN:].T
    tbl2 = _repack(wt, wtail)
    tbl = tbl2.reshape(_V, _D)
    idxt = x.astype(jnp.int32).T
    o4 = _gather(tbl, idxt)
    o5 = o4.reshape(_L, 8, _NW, 8, 128)
    return o5.transpose(2, 4, 0, 1, 3).reshape(_B, _L, _D)


# speedup vs baseline: 1.0836x; 1.0836x over previous
import functools
import math

import jax
import jax.numpy as jnp
from jax import lax
from jax.experimental import pallas as pl
from jax.experimental.pallas import tpu as pltpu
from jax.experimental.pallas import tpu_sc as plsc

_D = 64
_B = 4096
_L = 200
_V = 1000000
_N = _B * _L
_NC = 2
_NS = 16
_NW = _NC * _NS
_SCALE = math.sqrt(_D)

_VT = _V // 128
_VMAIN = _VT * 128
_TPB = 2
_QT = _VT // _TPB
_Q_PER_W = _QT // _NW
_Q_EXTRA = _QT - _Q_PER_W * _NW

_LPB = 2
_NBLK = _L // _LPB


def _iota16():
    return lax.iota(jnp.int32, 16)


def _k1_body(wt, wtail, tbl, in_v, out_v, tail_v,
             sg0, sg1, sg2, sg3, sw):
    wid = lax.axis_index("s") * _NC + lax.axis_index("c")
    start = wid * _Q_PER_W + jnp.minimum(wid, _Q_EXTRA)
    cnt = _Q_PER_W + jnp.where(wid < _Q_EXTRA, 1, 0)
    sg = (sg0, sg1, sg2, sg3)

    perm = []
    for t in range(_TPB):
        for j in range(8):
            v = j * 16 + _iota16()
            perm.append((t * 128 + j * 16, t * 64 + v // 2, (v % 2) * 64))

    def g_issue(q, b):
        pltpu.async_copy(
            wt.at[:, pl.ds(q * (128 * _TPB), 128 * _TPB)], in_v.at[b], sg[b]
        )

    def g_wait(q, b):
        pltpu.make_async_copy(
            wt.at[:, pl.ds(q * (128 * _TPB), 128 * _TPB)], in_v.at[b], sg[b]
        ).wait()

    def w_issue(q, b):
        pltpu.async_copy(
            out_v.at[b], tbl.at[pl.ds(q * (64 * _TPB), 64 * _TPB)], sw
        )

    def w_wait(q, b):
        pltpu.make_async_copy(
            out_v.at[b], tbl.at[pl.ds(q * (64 * _TPB), 64 * _TPB)], sw
        ).wait()

    for i in range(3):
        @pl.when(i < cnt)
        def _(i=i):
            g_issue(start + i, i)

    @pl.loop(0, (_Q_PER_W + 1 + 3) // 4)
    def _quad(p):
        for b in range(4):
            k = p * 4 + b

            @pl.when(k < cnt)
            def _(b=b, k=k):
                q = start + k
                g_wait(q, b)

                @pl.when(k >= 2)
                def _():
                    w_wait(q - 2, b % 2)

                @plsc.parallel_loop(0, 64, 1, unroll=2)
                def _tp(c, _b=b):
                    for off, d0, d1 in perm:
                        vals = in_v[_b, c, pl.ds(off, 16)] * _SCALE
                        plsc.store_scatter(
                            out_v.at[_b % 2], [d0, d1 + c], vals
                        )

                w_issue(q, b % 2)

                @pl.when(k + 3 < cnt)
                def _():
                    g_issue(q + 3, (b + 3) % 4)

    for i in range(2):
        @pl.when(cnt > i)
        def _(i=i):
            w_wait(start + cnt - 2 + i, (cnt - 2 + i) % 2)

    @pl.when(wid == _NW - 1)
    def _tail():
        pltpu.sync_copy(wtail, tail_v)

        @plsc.parallel_loop(0, 64, 1, unroll=2)
        def _tp(c):
            for j in range(4):
                v = j * 16 + _iota16()
                vals = tail_v[c, pl.ds(j * 16, 16)] * _SCALE
                plsc.store_scatter(
                    out_v.at[0], [v // 2, (v % 2) * 64 + c], vals
                )

        pltpu.sync_copy(
            out_v.at[0, pl.ds(0, 32)], tbl.at[pl.ds(_VMAIN // 2, 32)]
        )


@functools.partial(
    pl.kernel,
    out_type=jax.ShapeDtypeStruct((_V // 2, 128), jnp.float32),
    mesh=plsc.VectorSubcoreMesh(core_axis_name="c", subcore_axis_name="s"),
    scratch_types=[
        pltpu.VMEM((4, _D, 128 * _TPB), jnp.float32),
        pltpu.VMEM((2, 64 * _TPB, 128), jnp.float32),
        pltpu.VMEM((_D, _D), jnp.float32),
        pltpu.SemaphoreType.DMA,
        pltpu.SemaphoreType.DMA,
        pltpu.SemaphoreType.DMA,
        pltpu.SemaphoreType.DMA,
        pltpu.SemaphoreType.DMA,
    ],
    compiler_params=pltpu.CompilerParams(
        use_tc_tiling_on_sc=True, needs_layout_passes=False
    ),
)
def _repack(wt, wtail, tbl, in_v, out_v, tail_v, sg0, sg1, sg2, sg3, sw):
    _k1_body(wt, wtail, tbl, in_v, out_v, tail_v, sg0, sg1, sg2, sg3, sw)


def _k2_body(tbl, idxt, out, idx_v, rows_v, out_v,
             sg0, sg1, sg2, sg3, si0, si1, si2, si3, sw):
    wid = lax.axis_index("s") * _NC + lax.axis_index("c")
    sg = (sg0, sg1, sg2, sg3)
    si = (si0, si1, si2, si3)

    perm = []
    for u in range(_LPB):
        pu = []
        for j in range(4):
            c = j * 16 + _iota16()
            pu.append(u * 8192 + (c // 8) * 1024 + (c % 8) * 128)
        perm.append(pu)

    def i_issue(blk, b):
        pltpu.async_copy(
            idxt.at[pl.ds(blk * _LPB, _LPB), pl.ds(wid * 128, 128)],
            idx_v.at[b], si[b],
        )

    def i_wait(blk, b):
        pltpu.make_async_copy(
            idxt.at[pl.ds(blk * _LPB, _LPB), pl.ds(wid * 128, 128)],
            idx_v.at[b], si[b],
        ).wait()

    def g_issue(b):
        for u in range(_LPB):
            pltpu.async_copy(
                tbl.at[idx_v.at[b, u]],
                rows_v.at[b, pl.ds(u * 128, 128)],
                sg[b],
            )

    def g_wait(b):
        for u in range(_LPB):
            pltpu.make_async_copy(
                tbl.at[idx_v.at[b, u]],
                rows_v.at[b, pl.ds(u * 128, 128)],
                sg[b],
            ).wait()

    def w_issue(blk, b):
        l0 = blk * _LPB
        for u in range(_LPB):
            for ci in range(8):
                pltpu.async_copy(
                    out_v.at[b, pl.ds(u * 8192 + ci * 1024, 1024)],
                    out.at[l0 + u, ci, wid],
                    sw,
                )

    def w_wait(blk, b):
        l0 = blk * _LPB
        for u in range(_LPB):
            for ci in range(8):
                pltpu.make_async_copy(
                    out_v.at[b, pl.ds(u * 8192 + ci * 1024, 1024)],
                    out.at[l0 + u, ci, wid],
                    sw,
                ).wait()

    for i in range(3):
        i_issue(jnp.int32(i), i)
    i_wait(jnp.int32(0), 0)
    g_issue(0)
    i_wait(jnp.int32(1), 1)
    g_issue(1)

    @pl.loop(0, _NBLK // 4)
    def _quad(p):
        for b in range(4):
            blk = p * 4 + b

            g_wait(b)

            @pl.when(blk + 3 < _NBLK)
            def _(b=b, blk=blk):
                i_issue(blk + 3, (b + 3) % 4)

            @pl.when(blk + 2 < _NBLK)
            def _(b=b, blk=blk):
                i_wait(blk + 2, (b + 2) % 4)
                g_issue((b + 2) % 4)

            @pl.when(blk >= 2)
            def _(b=b, blk=blk):
                w_wait(blk - 2, b % 2)

            for u in range(_LPB):

                @plsc.parallel_loop(0, 128, 1, unroll=2)
                def _tp(rr, _b=b, _u=u):
                    for jj in range(4):
                        vals = rows_v[_b, _u * 128 + rr, pl.ds(jj * 16, 16)]
                        plsc.store_scatter(
                            out_v.at[_b % 2], [perm[_u][jj] + rr], vals
                        )

            w_issue(blk, b % 2)

    @pl.loop(0, 1)
    def _fin(_):
        for blkf in (_NBLK - 2, _NBLK - 1):
            w_wait(jnp.int32(blkf), blkf % 2)


@functools.partial(
    pl.kernel,
    out_type=jax.ShapeDtypeStruct((_L, 8, _NW, 1024), jnp.float32),
    mesh=plsc.VectorSubcoreMesh(core_axis_name="c", subcore_axis_name="s"),
    scratch_types=[
        pltpu.VMEM((4, _LPB, 128), jnp.int32),
        pltpu.VMEM((4, _LPB * 128, _D), jnp.float32),
        pltpu.VMEM((2, _LPB * 8192), jnp.float32),
        pltpu.SemaphoreType.DMA,
        pltpu.SemaphoreType.DMA,
        pltpu.SemaphoreType.DMA,
        pltpu.SemaphoreType.DMA,
        pltpu.SemaphoreType.DMA,
        pltpu.SemaphoreType.DMA,
        pltpu.SemaphoreType.DMA,
        pltpu.SemaphoreType.DMA,
        pltpu.SemaphoreType.DMA,
    ],
    compiler_params=pltpu.CompilerParams(
        use_tc_tiling_on_sc=False, needs_layout_passes=False
    ),
)
def _gather(tbl, idxt, out, idx_v, rows_v, out_v,
            sg0, sg1, sg2, sg3, si0, si1, si2, si3, sw):
    _k2_body(tbl, idxt, out, idx_v, rows_v, out_v,
             sg0, sg1, sg2, sg3, si0, si1, si2, si3, sw)


def kernel(x, embedding_weight):
    wt = embedding_weight.T
    wtail = embedding_weight[_VMAIN:].T
    tbl2 = _repack(wt, wtail)
    tbl = tbl2.reshape(_V, _D)
    idxt = x.astype(jnp.int32).T
    o4 = _gather(tbl, idxt)
    o5 = o4.reshape(_L, 8, _NW, 8, 128)
    return o5.transpose(2, 4, 0, 1, 3).reshape(_B, _L, _D)


# speedup vs baseline: 1.0865x; 1.0027x over previous
import functools
import math

import jax
import jax.numpy as jnp
from jax import lax
from jax.experimental import pallas as pl
from jax.experimental.pallas import tpu as pltpu
from jax.experimental.pallas import tpu_sc as plsc

_D = 64
_B = 4096
_L = 200
_V = 1000000
_N = _B * _L
_NC = 2
_NS = 16
_NW = _NC * _NS
_SCALE = math.sqrt(_D)

_VT = _V // 128
_VMAIN = _VT * 128
_TPB = 2
_QT = _VT // _TPB
_Q_PER_W = _QT // _NW
_Q_EXTRA = _QT - _Q_PER_W * _NW

_LPB = 2
_NBLK = _L // _LPB


def _iota16():
    return lax.iota(jnp.int32, 16)


def _k1_body(wt, wtail, tbl, in_v, out_v, tail_v,
             sg0, sg1, sg2, sg3, sw):
    wid = lax.axis_index("s") * _NC + lax.axis_index("c")
    start = wid * _Q_PER_W + jnp.minimum(wid, _Q_EXTRA)
    cnt = _Q_PER_W + jnp.where(wid < _Q_EXTRA, 1, 0)
    sg = (sg0, sg1, sg2, sg3)

    perm = []
    for t in range(_TPB):
        for j in range(8):
            v = j * 16 + _iota16()
            perm.append((t * 128 + j * 16, t * 64 + v // 2, (v % 2) * 64))

    def g_issue(q, b):
        pltpu.async_copy(
            wt.at[:, pl.ds(q * (128 * _TPB), 128 * _TPB)], in_v.at[b], sg[b]
        )

    def g_wait(q, b):
        pltpu.make_async_copy(
            wt.at[:, pl.ds(q * (128 * _TPB), 128 * _TPB)], in_v.at[b], sg[b]
        ).wait()

    def w_issue(q, b):
        pltpu.async_copy(
            out_v.at[b], tbl.at[pl.ds(q * (64 * _TPB), 64 * _TPB)], sw
        )

    def w_wait(q, b):
        pltpu.make_async_copy(
            out_v.at[b], tbl.at[pl.ds(q * (64 * _TPB), 64 * _TPB)], sw
        ).wait()

    for i in range(3):
        @pl.when(i < cnt)
        def _(i=i):
            g_issue(start + i, i)

    @pl.loop(0, (_Q_PER_W + 1 + 3) // 4)
    def _quad(p):
        for b in range(4):
            k = p * 4 + b

            @pl.when(k < cnt)
            def _(b=b, k=k):
                q = start + k
                g_wait(q, b)

                @pl.when(k >= 2)
                def _():
                    w_wait(q - 2, b % 2)

                @plsc.parallel_loop(0, 64, 1, unroll=2)
                def _tp(c, _b=b):
                    for off, d0, d1 in perm:
                        vals = in_v[_b, c, pl.ds(off, 16)] * _SCALE
                        plsc.store_scatter(
                            out_v.at[_b % 2], [d0, d1 + c], vals
                        )

                w_issue(q, b % 2)

                @pl.when(k + 3 < cnt)
                def _():
                    g_issue(q + 3, (b + 3) % 4)

    for i in range(2):
        @pl.when(cnt > i)
        def _(i=i):
            w_wait(start + cnt - 2 + i, (cnt - 2 + i) % 2)

    @pl.when(wid == _NW - 1)
    def _tail():
        pltpu.sync_copy(wtail, tail_v)

        @plsc.parallel_loop(0, 64, 1, unroll=2)
        def _tp(c):
            for j in range(4):
                v = j * 16 + _iota16()
                vals = tail_v[c, pl.ds(j * 16, 16)] * _SCALE
                plsc.store_scatter(
                    out_v.at[0], [v // 2, (v % 2) * 64 + c], vals
                )

        pltpu.sync_copy(
            out_v.at[0, pl.ds(0, 32)], tbl.at[pl.ds(_VMAIN // 2, 32)]
        )


@functools.partial(
    pl.kernel,
    out_type=jax.ShapeDtypeStruct((_V // 2, 128), jnp.float32),
    mesh=plsc.VectorSubcoreMesh(core_axis_name="c", subcore_axis_name="s"),
    scratch_types=[
        pltpu.VMEM((4, _D, 128 * _TPB), jnp.float32),
        pltpu.VMEM((2, 64 * _TPB, 128), jnp.float32),
        pltpu.VMEM((_D, _D), jnp.float32),
        pltpu.SemaphoreType.DMA,
        pltpu.SemaphoreType.DMA,
        pltpu.SemaphoreType.DMA,
        pltpu.SemaphoreType.DMA,
        pltpu.SemaphoreType.DMA,
    ],
    compiler_params=pltpu.CompilerParams(
        use_tc_tiling_on_sc=True, needs_layout_passes=False
    ),
)
def _repack(wt, wtail, tbl, in_v, out_v, tail_v, sg0, sg1, sg2, sg3, sw):
    _k1_body(wt, wtail, tbl, in_v, out_v, tail_v, sg0, sg1, sg2, sg3, sw)


def _k2_body(tbl, idxt, out, idx_v, rows_v, out_v,
             sg0, sg1, sg2, sg3, si0, si1, si2, si3, sw):
    wid = lax.axis_index("s") * _NC + lax.axis_index("c")
    sg = (sg0, sg1, sg2, sg3)
    si = (si0, si1, si2, si3)

    perm = []
    for u in range(_LPB):
        pu = []
        for j in range(4):
            c = j * 16 + _iota16()
            pu.append((u * 8 + c // 8, (c % 8) * 128))
        perm.append(pu)

    def i_issue(blk, b):
        pltpu.async_copy(
            idxt.at[pl.ds(blk * _LPB, _LPB), pl.ds(wid * 128, 128)],
            idx_v.at[b], si[b],
        )

    def i_wait(blk, b):
        pltpu.make_async_copy(
            idxt.at[pl.ds(blk * _LPB, _LPB), pl.ds(wid * 128, 128)],
            idx_v.at[b], si[b],
        ).wait()

    def g_issue(b):
        for u in range(_LPB):
            pltpu.async_copy(
                tbl.at[idx_v.at[b, u]],
                rows_v.at[b, pl.ds(u * 128, 128)],
                sg[b],
            )

    def g_wait(b):
        pltpu.make_async_copy(tbl.at[pl.ds(0, _LPB * 128)], rows_v.at[b], sg[b]).wait()

    def w_issue(blk, b):
        l0 = blk * _LPB
        for u in range(_LPB):
            for ci in range(8):
                pltpu.async_copy(
                    out_v.at[b, u * 8 + ci], out.at[l0 + u, ci, wid], sw
                )

    def w_wait(b):
        pltpu.make_async_copy(
            out.at[0, 0, pl.ds(0, 16)], out_v.at[b], sw
        ).wait()

    for i in range(3):
        i_issue(jnp.int32(i), i)
    i_wait(jnp.int32(0), 0)
    g_issue(0)
    i_wait(jnp.int32(1), 1)
    g_issue(1)

    @pl.loop(0, _NBLK // 4)
    def _quad(p):
        for b in range(4):
            blk = p * 4 + b

            g_wait(b)

            @pl.when(blk + 3 < _NBLK)
            def _(b=b, blk=blk):
                i_issue(blk + 3, (b + 3) % 4)

            @pl.when(blk + 2 < _NBLK)
            def _(b=b, blk=blk):
                i_wait(blk + 2, (b + 2) % 4)
                g_issue((b + 2) % 4)

            @pl.when(blk >= 2)
            def _(b=b):
                w_wait(b % 2)

            for u in range(_LPB):

                @plsc.parallel_loop(0, 128, 1, unroll=2)
                def _tp(rr, _b=b, _u=u):
                    for jj in range(4):
                        d0, d1 = perm[_u][jj]
                        vals = rows_v[_b, _u * 128 + rr, pl.ds(jj * 16, 16)]
                        plsc.store_scatter(
                            out_v.at[_b % 2], [d0, d1 + rr], vals
                        )

            w_issue(blk, b % 2)

    @pl.loop(0, 1)
    def _fin(_):
        for blkf in (_NBLK - 2, _NBLK - 1):
            w_wait(blkf % 2)


@functools.partial(
    pl.kernel,
    out_type=jax.ShapeDtypeStruct((_L, 8, _NW, 1024), jnp.float32),
    mesh=plsc.VectorSubcoreMesh(core_axis_name="c", subcore_axis_name="s"),
    scratch_types=[
        pltpu.VMEM((4, _LPB, 128), jnp.int32),
        pltpu.VMEM((4, _LPB * 128, _D), jnp.float32),
        pltpu.VMEM((2, _LPB * 8, 1024), jnp.float32),
        pltpu.SemaphoreType.DMA,
        pltpu.SemaphoreType.DMA,
        pltpu.SemaphoreType.DMA,
        pltpu.SemaphoreType.DMA,
        pltpu.SemaphoreType.DMA,
        pltpu.SemaphoreType.DMA,
        pltpu.SemaphoreType.DMA,
        pltpu.SemaphoreType.DMA,
        pltpu.SemaphoreType.DMA,
    ],
    compiler_params=pltpu.CompilerParams(
        use_tc_tiling_on_sc=False, needs_layout_passes=False
    ),
)
def _gather(tbl, idxt, out, idx_v, rows_v, out_v,
            sg0, sg1, sg2, sg3, si0, si1, si2, si3, sw):
    _k2_body(tbl, idxt, out, idx_v, rows_v, out_v,
             sg0, sg1, sg2, sg3, si0, si1, si2, si3, sw)


def kernel(x, embedding_weight):
    wt = embedding_weight.T
    wtail = embedding_weight[_VMAIN:].T
    tbl2 = _repack(wt, wtail)
    tbl = tbl2.reshape(_V, _D)
    idxt = x.astype(jnp.int32).T
    o4 = _gather(tbl, idxt)
    o5 = o4.reshape(_L, 8, _NW, 8, 128)
    return o5.transpose(2, 4, 0, 1, 3).reshape(_B, _L, _D)


# speedup vs baseline: 1.0894x; 1.0026x over previous
import functools
import math

import jax
import jax.numpy as jnp
from jax import lax
from jax.experimental import pallas as pl
from jax.experimental.pallas import tpu as pltpu
from jax.experimental.pallas import tpu_sc as plsc

_D = 64
_B = 4096
_L = 200
_V = 1000000
_N = _B * _L
_NC = 2
_NS = 16
_NW = _NC * _NS
_SCALE = math.sqrt(_D)

_VT = _V // 128
_VMAIN = _VT * 128
_TPB = 3
_QT = _VT // _TPB
_Q_PER_W = _QT // _NW
_Q_EXTRA = _QT - _Q_PER_W * _NW

_LPB = 2
_NBLK = _L // _LPB


def _iota16():
    return lax.iota(jnp.int32, 16)


def _k1_body(wt, wtail, tbl, in_v, out_v, tail_v, sg0, sg1, sw):
    wid = lax.axis_index("s") * _NC + lax.axis_index("c")
    start = wid * _Q_PER_W + jnp.minimum(wid, _Q_EXTRA)
    cnt = _Q_PER_W + jnp.where(wid < _Q_EXTRA, 1, 0)
    sg = (sg0, sg1)

    perm = []
    for t in range(_TPB):
        for j in range(8):
            v = j * 16 + _iota16()
            perm.append((t * 128 + j * 16, t * 64 + v // 2, (v % 2) * 64))

    def g_issue(q, b):
        for i in range(8):
            pltpu.async_copy(
                wt.at[pl.ds(i * 8, 8), pl.ds(q * (128 * _TPB), 128 * _TPB)],
                in_v.at[b, pl.ds(i * 8, 8)],
                sg[b],
            )

    def g_wait(b):
        pltpu.make_async_copy(
            wt.at[:, pl.ds(0, 128 * _TPB)], in_v.at[b], sg[b]
        ).wait()

    def w_issue(q, b):
        pltpu.async_copy(
            out_v.at[b], tbl.at[pl.ds(q * (64 * _TPB), 64 * _TPB)], sw
        )

    def w_wait(b):
        pltpu.make_async_copy(
            tbl.at[pl.ds(0, 64 * _TPB)], out_v.at[b], sw
        ).wait()

    g_issue(start, 0)

    @pl.loop(0, (_Q_PER_W + 1 + 1) // 2 + 1)
    def _pairk(p):
        for b in range(2):
            k = p * 2 + b

            @pl.when(k < cnt)
            def _(b=b, k=k):
                q = start + k
                g_wait(b)

                @pl.when(k + 1 < cnt)
                def _():
                    g_issue(q + 1, 1 - b)

                @pl.when(k >= 2)
                def _():
                    w_wait(b)

                @plsc.parallel_loop(0, 64, 1, unroll=2)
                def _tp(c, _b=b):
                    for off, d0, d1 in perm:
                        vals = in_v[_b, c, pl.ds(off, 16)] * _SCALE
                        plsc.store_scatter(out_v.at[_b], [d0, d1 + c], vals)

                w_issue(q, b)

    for i in range(2):
        @pl.when(cnt > i)
        def _(i=i):
            w_wait((cnt - 2 + i) % 2)

    @pl.when(wid == _NW - 1)
    def _tail():
        pltpu.sync_copy(wtail, tail_v)

        @plsc.parallel_loop(0, 64, 1, unroll=2)
        def _tp(c):
            for j in range(4):
                v = j * 16 + _iota16()
                vals = tail_v[c, pl.ds(j * 16, 16)] * _SCALE
                plsc.store_scatter(
                    out_v.at[0], [v // 2, (v % 2) * 64 + c], vals
                )

        pltpu.sync_copy(
            out_v.at[0, pl.ds(0, 32)], tbl.at[pl.ds(_VMAIN // 2, 32)]
        )


@functools.partial(
    pl.kernel,
    out_type=jax.ShapeDtypeStruct((_V // 2, 128), jnp.float32),
    mesh=plsc.VectorSubcoreMesh(core_axis_name="c", subcore_axis_name="s"),
    scratch_types=[
        pltpu.VMEM((2, _D, 128 * _TPB), jnp.float32),
        pltpu.VMEM((2, 64 * _TPB, 128), jnp.float32),
        pltpu.VMEM((_D, _D), jnp.float32),
        pltpu.SemaphoreType.DMA,
        pltpu.SemaphoreType.DMA,
        pltpu.SemaphoreType.DMA,
    ],
    compiler_params=pltpu.CompilerParams(
        use_tc_tiling_on_sc=True, needs_layout_passes=False
    ),
)
def _repack(wt, wtail, tbl, in_v, out_v, tail_v, sg0, sg1, sw):
    _k1_body(wt, wtail, tbl, in_v, out_v, tail_v, sg0, sg1, sw)


def _k2_body(tbl, idxt, out, idx_v, rows_v, out_v,
             sg0, sg1, sg2, sg3, si0, si1, si2, si3, sw):
    wid = lax.axis_index("s") * _NC + lax.axis_index("c")
    sg = (sg0, sg1, sg2, sg3)
    si = (si0, si1, si2, si3)

    perm = []
    for u in range(_LPB):
        pu = []
        for j in range(4):
            c = j * 16 + _iota16()
            pu.append((u * 8 + c // 8, (c % 8) * 128))
        perm.append(pu)

    def i_issue(blk, b):
        pltpu.async_copy(
            idxt.at[pl.ds(blk * _LPB, _LPB), pl.ds(wid * 128, 128)],
            idx_v.at[b], si[b],
        )

    def i_wait(blk, b):
        pltpu.make_async_copy(
            idxt.at[pl.ds(blk * _LPB, _LPB), pl.ds(wid * 128, 128)],
            idx_v.at[b], si[b],
        ).wait()

    def g_issue(b):
        for u in range(_LPB):
            pltpu.async_copy(
                tbl.at[idx_v.at[b, u]],
                rows_v.at[b, pl.ds(u * 128, 128)],
                sg[b],
            )

    def g_wait(b):
        pltpu.make_async_copy(tbl.at[pl.ds(0, _LPB * 128)], rows_v.at[b], sg[b]).wait()

    def w_issue(blk, b):
        l0 = blk * _LPB
        for u in range(_LPB):
            pltpu.async_copy(
                out_v.at[b, pl.ds(u * 8, 8)], out.at[l0 + u, :, wid], sw
            )

    def w_wait(b):
        pltpu.make_async_copy(
            out.at[0, 0, pl.ds(0, 16)], out_v.at[b], sw
        ).wait()

    for i in range(3):
        i_issue(jnp.int32(i), i)
    i_wait(jnp.int32(0), 0)
    g_issue(0)
    i_wait(jnp.int32(1), 1)
    g_issue(1)

    @pl.loop(0, _NBLK // 4)
    def _quad(p):
        for b in range(4):
            blk = p * 4 + b

            g_wait(b)

            @pl.when(blk + 3 < _NBLK)
            def _(b=b, blk=blk):
                i_issue(blk + 3, (b + 3) % 4)

            @pl.when(blk + 2 < _NBLK)
            def _(b=b, blk=blk):
                i_wait(blk + 2, (b + 2) % 4)
                g_issue((b + 2) % 4)

            @pl.when(blk >= 2)
            def _(b=b):
                w_wait(b % 2)

            for u in range(_LPB):

                @plsc.parallel_loop(0, 128, 1, unroll=2)
                def _tp(rr, _b=b, _u=u):
                    for jj in range(4):
                        d0, d1 = perm[_u][jj]
                        vals = rows_v[_b, _u * 128 + rr, pl.ds(jj * 16, 16)]
                        plsc.store_scatter(
                            out_v.at[_b % 2], [d0, d1 + rr], vals
                        )

            w_issue(blk, b % 2)

    @pl.loop(0, 1)
    def _fin(_):
        for blkf in (_NBLK - 2, _NBLK - 1):
            w_wait(blkf % 2)


@functools.partial(
    pl.kernel,
    out_type=jax.ShapeDtypeStruct((_L, 8, _NW, 1024), jnp.float32),
    mesh=plsc.VectorSubcoreMesh(core_axis_name="c", subcore_axis_name="s"),
    scratch_types=[
        pltpu.VMEM((4, _LPB, 128), jnp.int32),
        pltpu.VMEM((4, _LPB * 128, _D), jnp.float32),
        pltpu.VMEM((2, _LPB * 8, 1024), jnp.float32),
        pltpu.SemaphoreType.DMA,
        pltpu.SemaphoreType.DMA,
        pltpu.SemaphoreType.DMA,
        pltpu.SemaphoreType.DMA,
        pltpu.SemaphoreType.DMA,
        pltpu.SemaphoreType.DMA,
        pltpu.SemaphoreType.DMA,
        pltpu.SemaphoreType.DMA,
        pltpu.SemaphoreType.DMA,
    ],
    compiler_params=pltpu.CompilerParams(
        use_tc_tiling_on_sc=False, needs_layout_passes=False
    ),
)
def _gather(tbl, idxt, out, idx_v, rows_v, out_v,
            sg0, sg1, sg2, sg3, si0, si1, si2, si3, sw):
    _k2_body(tbl, idxt, out, idx_v, rows_v, out_v,
             sg0, sg1, sg2, sg3, si0, si1, si2, si3, sw)


def kernel(x, embedding_weight):
    wt = embedding_weight.T
    wtail = embedding_weight[_VMAIN:].T
    tbl2 = _repack(wt, wtail)
    tbl = tbl2.reshape(_V, _D)
    idxt = x.astype(jnp.int32).T
    o4 = _gather(tbl, idxt)
    o5 = o4.reshape(_L, 8, _NW, 8, 128)
    return o5.transpose(2, 4, 0, 1, 3).reshape(_B, _L, _D)


# speedup vs baseline: 1.6729x; 1.5356x over previous
import functools
import math

import jax
import jax.numpy as jnp
from jax import lax
from jax.experimental import pallas as pl
from jax.experimental.pallas import tpu as pltpu
from jax.experimental.pallas import tpu_sc as plsc

_D = 64
_B = 4096
_L = 200
_V = 1000000
_N = _B * _L
_NC = 2
_NS = 16
_NW = _NC * _NS
_SCALE = math.sqrt(_D)

_VT = _V // 128
_VMAIN = _VT * 128
_TPB = 2
_QT = _VT // _TPB
_Q_PER_W = _QT // _NW
_Q_EXTRA = _QT - _Q_PER_W * _NW

_LPB = 2
_NBLK = _L // _LPB


def _iota16():
    return lax.iota(jnp.int32, 16)


def _k1_body(wt, wtail, tbl, in_v, out_v, tail_v, sg0, sg1, sw):
    wid = lax.axis_index("s") * _NC + lax.axis_index("c")
    start = wid * _Q_PER_W + jnp.minimum(wid, _Q_EXTRA)
    cnt = _Q_PER_W + jnp.where(wid < _Q_EXTRA, 1, 0)
    sg = (sg0, sg1)

    perm = []
    for t in range(_TPB):
        for j in range(8):
            v = j * 16 + _iota16()
            perm.append((t * 128 + j * 16, t * 64 + v // 2, (v % 2) * 64))

    def g_issue(q, b):
        for i in range(8):
            pltpu.async_copy(
                wt.at[pl.ds(i * 8, 8), pl.ds(q * (128 * _TPB), 128 * _TPB)],
                in_v.at[b, pl.ds(i * 8, 8)],
                sg[b],
            )

    def g_wait(b):
        pltpu.make_async_copy(
            wt.at[:, pl.ds(0, 128 * _TPB)], in_v.at[b], sg[b]
        ).wait()

    def w_issue(q, b):
        pltpu.async_copy(
            out_v.at[b, :, pl.ds(0, 128)],
            tbl.at[pl.ds(q * (64 * _TPB), 64 * _TPB)],
            sw,
        )

    def w_wait(b):
        pltpu.make_async_copy(
            tbl.at[pl.ds(0, 64 * _TPB)], out_v.at[b, :, pl.ds(0, 128)], sw
        ).wait()

    g_issue(start, 0)

    @pl.loop(0, (_Q_PER_W + 1 + 1) // 2 + 1)
    def _pairk(p):
        for b in range(2):
            k = p * 2 + b

            @pl.when(k < cnt)
            def _(b=b, k=k):
                q = start + k
                g_wait(b)

                @pl.when(k + 1 < cnt)
                def _():
                    g_issue(q + 1, 1 - b)

                @pl.when(k >= 2)
                def _():
                    w_wait(b)

                @plsc.parallel_loop(0, 64, 1, unroll=2)
                def _tp(c, _b=b):
                    for off, d0, d1 in perm:
                        vals = in_v[_b, c, pl.ds(off, 16)] * _SCALE
                        plsc.store_scatter(out_v.at[_b], [d0, d1 + c], vals)

                w_issue(q, b)

    for i in range(2):
        @pl.when(cnt > i)
        def _(i=i):
            w_wait((cnt - 2 + i) % 2)

    @pl.when(wid == _NW - 1)
    def _tail():
        pltpu.sync_copy(wtail, tail_v)

        @plsc.parallel_loop(0, 64, 1, unroll=2)
        def _tp(c):
            for j in range(4):
                v = j * 16 + _iota16()
                vals = tail_v[c, pl.ds(j * 16, 16)] * _SCALE
                plsc.store_scatter(
                    out_v.at[0], [v // 2, (v % 2) * 64 + c], vals
                )

        pltpu.sync_copy(
            out_v.at[0, pl.ds(0, 32), pl.ds(0, 128)],
            tbl.at[pl.ds(_VMAIN // 2, 32)],
        )


@functools.partial(
    pl.kernel,
    out_type=jax.ShapeDtypeStruct((_V // 2, 128), jnp.float32),
    mesh=plsc.VectorSubcoreMesh(core_axis_name="c", subcore_axis_name="s"),
    scratch_types=[
        pltpu.VMEM((2, _D, 128 * _TPB), jnp.float32),
        pltpu.VMEM((2, 64 * _TPB, 137), jnp.float32),
        pltpu.VMEM((_D, _D), jnp.float32),
        pltpu.SemaphoreType.DMA,
        pltpu.SemaphoreType.DMA,
        pltpu.SemaphoreType.DMA,
    ],
    compiler_params=pltpu.CompilerParams(
        use_tc_tiling_on_sc=True, needs_layout_passes=False
    ),
)
def _repack(wt, wtail, tbl, in_v, out_v, tail_v, sg0, sg1, sw):
    _k1_body(wt, wtail, tbl, in_v, out_v, tail_v, sg0, sg1, sw)


def _k2_body(tbl, idxt, out, idx_v, rows_v, out_v,
             sg0, sg1, sg2, sg3, si0, si1, si2, si3, sw):
    wid = lax.axis_index("s") * _NC + lax.axis_index("c")
    sg = (sg0, sg1, sg2, sg3)
    si = (si0, si1, si2, si3)

    perm = []
    for u in range(_LPB):
        pu = []
        for j in range(4):
            c = j * 16 + _iota16()
            pu.append((u * 8 + c // 8, c % 8))
        perm.append(pu)

    def i_issue(blk, b):
        pltpu.async_copy(
            idxt.at[pl.ds(blk * _LPB, _LPB), pl.ds(wid * 128, 128)],
            idx_v.at[b], si[b],
        )

    def i_wait(blk, b):
        pltpu.make_async_copy(
            idxt.at[pl.ds(blk * _LPB, _LPB), pl.ds(wid * 128, 128)],
            idx_v.at[b], si[b],
        ).wait()

    def g_issue(b):
        for u in range(_LPB):
            pltpu.async_copy(
                tbl.at[idx_v.at[b, u]],
                rows_v.at[b, pl.ds(u * 128, 128)],
                sg[b],
            )

    def g_wait(b):
        pltpu.make_async_copy(tbl.at[pl.ds(0, _LPB * 128)], rows_v.at[b], sg[b]).wait()

    def w_issue(blk, b):
        l0 = blk * _LPB
        for u in range(_LPB):
            pltpu.async_copy(
                out_v.at[b, pl.ds(u * 8, 8), :, pl.ds(0, 128)],
                out.at[l0 + u, :, wid],
                sw,
            )

    def w_wait(b):
        for u in range(_LPB):
            pltpu.make_async_copy(
                out.at[0, :, 0],
                out_v.at[b, pl.ds(u * 8, 8), :, pl.ds(0, 128)],
                sw,
            ).wait()

    for i in range(3):
        i_issue(jnp.int32(i), i)
    i_wait(jnp.int32(0), 0)
    g_issue(0)
    i_wait(jnp.int32(1), 1)
    g_issue(1)

    @pl.loop(0, _NBLK // 4)
    def _quad(p):
        for b in range(4):
            blk = p * 4 + b

            g_wait(b)

            @pl.when(blk + 3 < _NBLK)
            def _(b=b, blk=blk):
                i_issue(blk + 3, (b + 3) % 4)

            @pl.when(blk + 2 < _NBLK)
            def _(b=b, blk=blk):
                i_wait(blk + 2, (b + 2) % 4)
                g_issue((b + 2) % 4)

            @pl.when(blk >= 2)
            def _(b=b):
                w_wait(b % 2)

            for u in range(_LPB):

                @plsc.parallel_loop(0, 128, 1, unroll=2)
                def _tp(rr, _b=b, _u=u):
                    for jj in range(4):
                        d0, d1 = perm[_u][jj]
                        vals = rows_v[_b, _u * 128 + rr, pl.ds(jj * 16, 16)]
                        plsc.store_scatter(
                            out_v.at[_b % 2],
                            [d0, d1, jnp.zeros((16,), jnp.int32) + rr],
                            vals,
                        )

            w_issue(blk, b % 2)

    @pl.loop(0, 1)
    def _fin(_):
        for blkf in (_NBLK - 2, _NBLK - 1):
            w_wait(blkf % 2)


@functools.partial(
    pl.kernel,
    out_type=jax.ShapeDtypeStruct((_L, 8, _NW, 8, 128), jnp.float32),
    mesh=plsc.VectorSubcoreMesh(core_axis_name="c", subcore_axis_name="s"),
    scratch_types=[
        pltpu.VMEM((4, _LPB, 128), jnp.int32),
        pltpu.VMEM((4, _LPB * 128, _D), jnp.float32),
        pltpu.VMEM((2, _LPB * 8, 8, 137), jnp.float32),
        pltpu.SemaphoreType.DMA,
        pltpu.SemaphoreType.DMA,
        pltpu.SemaphoreType.DMA,
        pltpu.SemaphoreType.DMA,
        pltpu.SemaphoreType.DMA,
        pltpu.SemaphoreType.DMA,
        pltpu.SemaphoreType.DMA,
        pltpu.SemaphoreType.DMA,
        pltpu.SemaphoreType.DMA,
    ],
    compiler_params=pltpu.CompilerParams(
        use_tc_tiling_on_sc=False, needs_layout_passes=False
    ),
)
def _gather(tbl, idxt, out, idx_v, rows_v, out_v,
            sg0, sg1, sg2, sg3, si0, si1, si2, si3, sw):
    _k2_body(tbl, idxt, out, idx_v, rows_v, out_v,
             sg0, sg1, sg2, sg3, si0, si1, si2, si3, sw)


def kernel(x, embedding_weight):
    wt = embedding_weight.T
    wtail = embedding_weight[_VMAIN:].T
    tbl2 = _repack(wt, wtail)
    tbl = tbl2.reshape(_V, _D)
    idxt = x.astype(jnp.int32).T
    o5 = _gather(tbl, idxt)
    return o5.transpose(2, 4, 0, 1, 3).reshape(_B, _L, _D)
